# baseline (speedup 1.0000x reference)
import jax
import jax.numpy as jnp
from jax import lax
from jax.experimental import pallas as pl
from jax.experimental.pallas import tpu as pltpu
from jax.experimental.pallas import tpu_sc as plsc

H = 128
N = 10000
E = 160000
EPS = 1e-5

NC = 2
NS = 16
K = 128
EPS_PER_SUB = 10240
HALF = EPS_PER_SUB // 2
HCH = HALF // K
EP = NS * EPS_PER_SUB
NP = 10240
SLAB = NP // NS


def _make_sc_body(with_counts):
    def body(*refs):
        if with_counts:
            (h_u, h_i, s_ui, d_ui, s_iu, d_iu, z128, z1, ones1,
             agg_i, cnt_i, agg_u, cnt_u,
             acc, accc, sidx, didx, rows0, rows1, onesb, semA, semB) = refs
        else:
            (h_u, h_i, s_ui, d_ui, s_iu, d_iu, z128,
             agg_i, agg_u,
             acc, sidx, didx, rows0, rows1, semA, semB) = refs
        c = lax.axis_index("c")
        s = lax.axis_index("s")

        def do_side(hsrc, sflat, dflat, agg_out, cnt_out):
            pltpu.sync_copy(z128, acc.at[pl.ds(s * SLAB, SLAB)])
            if with_counts:
                pltpu.sync_copy(z1, accc.at[pl.ds(s * SLAB, SLAB)])
                pltpu.sync_copy(ones1, onesb)
            base = s * EPS_PER_SUB
            plsc.subcore_barrier()

            def scat(j, rows):
                dsl = didx.at[pl.ds(j * K, K)]
                pltpu.sync_copy(rows, acc.at[dsl], add=True)
                if with_counts:
                    pltpu.sync_copy(onesb, accc.at[dsl], add=True)

            def gath(j, rows, sem):
                pltpu.async_copy(hsrc.at[sidx.at[pl.ds(j * K, K)]], rows, sem)

            def wait(rows, sem):
                pltpu.make_async_copy(hsrc.at[sidx.at[pl.ds(0, K)]], rows,
                                      sem).wait()

            for g in range(2):
                pltpu.sync_copy(sflat.at[pl.ds(base + g * HALF, HALF)], sidx)
                pltpu.sync_copy(dflat.at[pl.ds(base + g * HALF, HALF)], didx)
                gath(0, rows0, semA)

                def pair(j2, carry):
                    j = 2 * j2
                    gath(j + 1, rows1, semB)
                    wait(rows0, semA)
                    scat(j, rows0)
                    gath(lax.rem(j + 2, HCH), rows0, semA)
                    wait(rows1, semB)
                    scat(j + 1, rows1)
                    return carry

                lax.fori_loop(0, HCH // 2, pair, 0)
                wait(rows0, semA)

            plsc.subcore_barrier()
            pltpu.sync_copy(acc.at[pl.ds(s * SLAB, SLAB)],
                            agg_out.at[pl.ds(s * SLAB, SLAB)])
            if with_counts:
                pltpu.sync_copy(accc.at[pl.ds(s * SLAB, SLAB)],
                                cnt_out.at[pl.ds(s * SLAB, SLAB)])

        @pl.when(c == 0)
        def _():
            do_side(h_u, s_ui, d_ui, agg_i, cnt_i if with_counts else None)

        @pl.when(c == 1)
        def _():
            do_side(h_i, s_iu, d_iu, agg_u, cnt_u if with_counts else None)

    return body


@jax.jit
def _sc_agg_counts(h_u, h_i, s_ui, d_ui, s_iu, d_iu, z128, z1, ones1):
    mesh = plsc.VectorSubcoreMesh(core_axis_name="c", subcore_axis_name="s",
                                  num_cores=NC, num_subcores=NS)
    f = pl.kernel(
        _make_sc_body(True),
        out_type=(
            jax.ShapeDtypeStruct((NP, H), jnp.float32),
            jax.ShapeDtypeStruct((NP,), jnp.float32),
            jax.ShapeDtypeStruct((NP, H), jnp.float32),
            jax.ShapeDtypeStruct((NP,), jnp.float32),
        ),
        mesh=mesh,
        scratch_types=[
            pltpu.VMEM_SHARED((NP, H), jnp.float32),
            pltpu.VMEM_SHARED((NP,), jnp.float32),
            pltpu.VMEM((HALF,), jnp.int32),
            pltpu.VMEM((HALF,), jnp.int32),
            pltpu.VMEM((K, H), jnp.float32),
            pltpu.VMEM((K, H), jnp.float32),
            pltpu.VMEM((K,), jnp.float32),
            pltpu.SemaphoreType.DMA,
            pltpu.SemaphoreType.DMA,
        ],
    )
    return f(h_u, h_i, s_ui, d_ui, s_iu, d_iu, z128, z1, ones1)


@jax.jit
def _sc_agg_plain(h_u, h_i, s_ui, d_ui, s_iu, d_iu, z128):
    mesh = plsc.VectorSubcoreMesh(core_axis_name="c", subcore_axis_name="s",
                                  num_cores=NC, num_subcores=NS)
    f = pl.kernel(
        _make_sc_body(False),
        out_type=(
            jax.ShapeDtypeStruct((NP, H), jnp.float32),
            jax.ShapeDtypeStruct((NP, H), jnp.float32),
        ),
        mesh=mesh,
        scratch_types=[
            pltpu.VMEM_SHARED((NP, H), jnp.float32),
            pltpu.VMEM((HALF,), jnp.int32),
            pltpu.VMEM((HALF,), jnp.int32),
            pltpu.VMEM((K, H), jnp.float32),
            pltpu.VMEM((K, H), jnp.float32),
            pltpu.SemaphoreType.DMA,
            pltpu.SemaphoreType.DMA,
        ],
    )
    return f(h_u, h_i, s_ui, d_ui, s_iu, d_iu, z128)


def _proj_body(xu, wu, bu, xi, wi, bi, hu, hi):
    hu[...] = jnp.dot(xu[...], wu[...],
                      preferred_element_type=jnp.float32) + bu[...]
    hi[...] = jnp.dot(xi[...], wi[...],
                      preferred_element_type=jnp.float32) + bi[...]


@jax.jit
def _proj(xu, wu, bu, xi, wi, bi):
    return pl.pallas_call(
        _proj_body,
        out_shape=(jax.ShapeDtypeStruct((N, H), jnp.float32),
                   jax.ShapeDtypeStruct((N, H), jnp.float32)),
    )(xu, wu, bu, xi, wi, bi)


def _layer_side(agg, cnt, h, wl, bl, wr, g, b):
    mean = agg[...][:N] / jnp.maximum(cnt[...], 1.0)
    x = (jnp.dot(mean, wl[...], preferred_element_type=jnp.float32) + bl[...]
         + jnp.dot(h[...], wr[...], preferred_element_type=jnp.float32))
    m = jnp.mean(x, axis=0, keepdims=True)
    v = jnp.mean((x - m) * (x - m), axis=0, keepdims=True)
    y = g[...] * (x - m) * lax.rsqrt(v + EPS) + b[...]
    return jnp.maximum(y, 0.0) + h[...]


def _layer_body(agg_i, cnt_i, hi, wl_ui, bl_ui, wr_ui, gi, bi,
                agg_u, cnt_u, hu, wl_iu, bl_iu, wr_iu, gu, bu,
                hi_new, hu_new):
    hi_new[...] = _layer_side(agg_i, cnt_i, hi, wl_ui, bl_ui, wr_ui, gi, bi)
    hu_new[...] = _layer_side(agg_u, cnt_u, hu, wl_iu, bl_iu, wr_iu, gu, bu)


@jax.jit
def _layer(agg_i, cnt_i, hi, wl_ui, bl_ui, wr_ui, gi, bi,
           agg_u, cnt_u, hu, wl_iu, bl_iu, wr_iu, gu, bu):
    return pl.pallas_call(
        _layer_body,
        out_shape=(jax.ShapeDtypeStruct((N, H), jnp.float32),
                   jax.ShapeDtypeStruct((N, H), jnp.float32)),
    )(agg_i, cnt_i, hi, wl_ui, bl_ui, wr_ui, gi, bi,
      agg_u, cnt_u, hu, wl_iu, bl_iu, wr_iu, gu, bu)


def _pad_edges(ei):
    src = jnp.concatenate([ei[0], jnp.zeros((EP - E,), jnp.int32)])
    dst = jnp.concatenate([ei[1], jnp.full((EP - E,), NP - 1, jnp.int32)])
    return src, dst


def kernel(x_user, x_item, edge_index_user_item, edge_index_item_user,
           Wp_user, bp_user, Wp_item, bp_item,
           Wl0_ui, bl0_ui, Wr0_ui, Wl0_iu, bl0_iu, Wr0_iu,
           gamma0_user, beta0_user, gamma0_item, beta0_item,
           Wl1_ui, bl1_ui, Wr1_ui, Wl1_iu, bl1_iu, Wr1_iu,
           gamma1_user, beta1_user, gamma1_item, beta1_item):
    s_ui, d_ui = _pad_edges(edge_index_user_item)
    s_iu, d_iu = _pad_edges(edge_index_item_user)
    z128 = jnp.zeros((SLAB, H), jnp.float32)
    z1 = jnp.zeros((SLAB,), jnp.float32)
    ones1 = jnp.ones((K,), jnp.float32)

    r1 = lambda a: a.reshape(1, H)
    cnt2d = lambda cnt: cnt[:N].reshape(N, 1)
    h_u, h_i = _proj(x_user, Wp_user, r1(bp_user), x_item, Wp_item,
                     r1(bp_item))

    agg_i, cnt_i, agg_u, cnt_u = _sc_agg_counts(h_u, h_i, s_ui, d_ui,
                                                s_iu, d_iu, z128, z1, ones1)
    ci, cu = cnt2d(cnt_i), cnt2d(cnt_u)
    h_i, h_u = _layer(agg_i, ci, h_i, Wl0_ui, r1(bl0_ui), Wr0_ui,
                      r1(gamma0_item), r1(beta0_item),
                      agg_u, cu, h_u, Wl0_iu, r1(bl0_iu), Wr0_iu,
                      r1(gamma0_user), r1(beta0_user))

    agg_i, agg_u = _sc_agg_plain(h_u, h_i, s_ui, d_ui, s_iu, d_iu, z128)
    h_i, h_u = _layer(agg_i, ci, h_i, Wl1_ui, r1(bl1_ui), Wr1_ui,
                      r1(gamma1_item), r1(beta1_item),
                      agg_u, cu, h_u, Wl1_iu, r1(bl1_iu), Wr1_iu,
                      r1(gamma1_user), r1(beta1_user))
    return h_u, h_i


# speedup vs baseline: 2.1744x; 2.1744x over previous
import jax
import jax.numpy as jnp
from jax import lax
from jax.experimental import pallas as pl
from jax.experimental.pallas import tpu as pltpu
from jax.experimental.pallas import tpu_sc as plsc

H = 128
N = 10000
E = 160000
EPS = 1e-5

NC = 2
NS = 16
K = 80
EPS_PER_SUB = E // NS
CH = EPS_PER_SUB // K
NP = 10240
SLAB = NP // NS


def _make_sc_body(with_counts):
    def body(*refs):
        if with_counts:
            (h_u, h_i, s_ui, d_ui, s_iu, d_iu, z128, z1, ones1,
             agg_i, cnt_i, agg_u, cnt_u,
             acc, accc, sidx, didx, rows0, rows1, onesb, semA, semB) = refs
        else:
            (h_u, h_i, s_ui, d_ui, s_iu, d_iu, z128,
             agg_i, agg_u,
             acc, sidx, didx, rows0, rows1, semA, semB) = refs
        c = lax.axis_index("c")
        s = lax.axis_index("s")

        def do_side(hsrc, sflat, dflat, agg_out, cnt_out):
            pltpu.sync_copy(z128, acc.at[pl.ds(s * SLAB, SLAB)])
            if with_counts:
                pltpu.sync_copy(z1, accc.at[pl.ds(s * SLAB, SLAB)])
                pltpu.sync_copy(ones1, onesb)
            base = s * EPS_PER_SUB
            plsc.subcore_barrier()

            def scat(j, rows):
                dsl = didx.at[pl.ds(j * K, K)]
                pltpu.sync_copy(rows, acc.at[dsl], add=True)
                if with_counts:
                    pltpu.sync_copy(onesb, accc.at[dsl], add=True)

            def gath(j, rows, sem):
                pltpu.async_copy(hsrc.at[sidx.at[pl.ds(j * K, K)]], rows, sem)

            def wait(rows, sem):
                pltpu.make_async_copy(hsrc.at[sidx.at[pl.ds(0, K)]], rows,
                                      sem).wait()

            pltpu.sync_copy(sflat.at[pl.ds(base, EPS_PER_SUB)], sidx)
            pltpu.sync_copy(dflat.at[pl.ds(base, EPS_PER_SUB)], didx)
            gath(0, rows0, semA)

            def pair(j2, carry):
                j = 2 * j2
                gath(j + 1, rows1, semB)
                wait(rows0, semA)
                scat(j, rows0)
                gath(j + 2, rows0, semA)
                wait(rows1, semB)
                scat(j + 1, rows1)
                return carry

            lax.fori_loop(0, CH // 2, pair, 0)
            wait(rows0, semA)
            scat(CH - 1, rows0)

            plsc.subcore_barrier()
            pltpu.sync_copy(acc.at[pl.ds(s * SLAB, SLAB)],
                            agg_out.at[pl.ds(s * SLAB, SLAB)])
            if with_counts:
                pltpu.sync_copy(accc.at[pl.ds(s * SLAB, SLAB)],
                                cnt_out.at[pl.ds(s * SLAB, SLAB)])

        @pl.when(c == 0)
        def _():
            do_side(h_u, s_ui, d_ui, agg_i, cnt_i if with_counts else None)

        @pl.when(c == 1)
        def _():
            do_side(h_i, s_iu, d_iu, agg_u, cnt_u if with_counts else None)

    return body


@jax.jit
def _sc_agg_counts(h_u, h_i, s_ui, d_ui, s_iu, d_iu, z128, z1, ones1):
    mesh = plsc.VectorSubcoreMesh(core_axis_name="c", subcore_axis_name="s",
                                  num_cores=NC, num_subcores=NS)
    f = pl.kernel(
        _make_sc_body(True),
        out_type=(
            jax.ShapeDtypeStruct((NP, H), jnp.float32),
            jax.ShapeDtypeStruct((NP,), jnp.float32),
            jax.ShapeDtypeStruct((NP, H), jnp.float32),
            jax.ShapeDtypeStruct((NP,), jnp.float32),
        ),
        mesh=mesh,
        scratch_types=[
            pltpu.VMEM_SHARED((NP, H), jnp.float32),
            pltpu.VMEM_SHARED((NP,), jnp.float32),
            pltpu.VMEM((EPS_PER_SUB,), jnp.int32),
            pltpu.VMEM((EPS_PER_SUB,), jnp.int32),
            pltpu.VMEM((K, H), jnp.float32),
            pltpu.VMEM((K, H), jnp.float32),
            pltpu.VMEM((K,), jnp.float32),
            pltpu.SemaphoreType.DMA,
            pltpu.SemaphoreType.DMA,
        ],
    )
    return f(h_u, h_i, s_ui, d_ui, s_iu, d_iu, z128, z1, ones1)


@jax.jit
def _sc_agg_plain(h_u, h_i, s_ui, d_ui, s_iu, d_iu, z128):
    mesh = plsc.VectorSubcoreMesh(core_axis_name="c", subcore_axis_name="s",
                                  num_cores=NC, num_subcores=NS)
    f = pl.kernel(
        _make_sc_body(False),
        out_type=(
            jax.ShapeDtypeStruct((NP, H), jnp.float32),
            jax.ShapeDtypeStruct((NP, H), jnp.float32),
        ),
        mesh=mesh,
        scratch_types=[
            pltpu.VMEM_SHARED((NP, H), jnp.float32),
            pltpu.VMEM((EPS_PER_SUB,), jnp.int32),
            pltpu.VMEM((EPS_PER_SUB,), jnp.int32),
            pltpu.VMEM((K, H), jnp.float32),
            pltpu.VMEM((K, H), jnp.float32),
            pltpu.SemaphoreType.DMA,
            pltpu.SemaphoreType.DMA,
        ],
    )
    return f(h_u, h_i, s_ui, d_ui, s_iu, d_iu, z128)


def _proj_body(xu, wu, bu, xi, wi, bi, hu, hi):
    hu[...] = jnp.dot(xu[...], wu[...],
                      preferred_element_type=jnp.float32) + bu[...]
    hi[...] = jnp.dot(xi[...], wi[...],
                      preferred_element_type=jnp.float32) + bi[...]


@jax.jit
def _proj(xu, wu, bu, xi, wi, bi):
    return pl.pallas_call(
        _proj_body,
        out_shape=(jax.ShapeDtypeStruct((N, H), jnp.float32),
                   jax.ShapeDtypeStruct((N, H), jnp.float32)),
    )(xu, wu, bu, xi, wi, bi)


def _layer_side(agg, cnt, h, wl, bl, wr, g, b):
    mean = agg[...][:N] / jnp.maximum(cnt[...], 1.0)
    x = (jnp.dot(mean, wl[...], preferred_element_type=jnp.float32) + bl[...]
         + jnp.dot(h[...], wr[...], preferred_element_type=jnp.float32))
    m = jnp.mean(x, axis=0, keepdims=True)
    v = jnp.mean((x - m) * (x - m), axis=0, keepdims=True)
    y = g[...] * (x - m) * lax.rsqrt(v + EPS) + b[...]
    return jnp.maximum(y, 0.0) + h[...]


def _layer_body(agg_i, cnt_i, hi, wl_ui, bl_ui, wr_ui, gi, bi,
                agg_u, cnt_u, hu, wl_iu, bl_iu, wr_iu, gu, bu,
                hi_new, hu_new):
    hi_new[...] = _layer_side(agg_i, cnt_i, hi, wl_ui, bl_ui, wr_ui, gi, bi)
    hu_new[...] = _layer_side(agg_u, cnt_u, hu, wl_iu, bl_iu, wr_iu, gu, bu)


@jax.jit
def _layer(agg_i, cnt_i, hi, wl_ui, bl_ui, wr_ui, gi, bi,
           agg_u, cnt_u, hu, wl_iu, bl_iu, wr_iu, gu, bu):
    return pl.pallas_call(
        _layer_body,
        out_shape=(jax.ShapeDtypeStruct((N, H), jnp.float32),
                   jax.ShapeDtypeStruct((N, H), jnp.float32)),
    )(agg_i, cnt_i, hi, wl_ui, bl_ui, wr_ui, gi, bi,
      agg_u, cnt_u, hu, wl_iu, bl_iu, wr_iu, gu, bu)


def kernel(x_user, x_item, edge_index_user_item, edge_index_item_user,
           Wp_user, bp_user, Wp_item, bp_item,
           Wl0_ui, bl0_ui, Wr0_ui, Wl0_iu, bl0_iu, Wr0_iu,
           gamma0_user, beta0_user, gamma0_item, beta0_item,
           Wl1_ui, bl1_ui, Wr1_ui, Wl1_iu, bl1_iu, Wr1_iu,
           gamma1_user, beta1_user, gamma1_item, beta1_item):
    s_ui, d_ui = edge_index_user_item[0], edge_index_user_item[1]
    s_iu, d_iu = edge_index_item_user[0], edge_index_item_user[1]
    z128 = jnp.zeros((SLAB, H), jnp.float32)
    z1 = jnp.zeros((SLAB,), jnp.float32)
    ones1 = jnp.ones((K,), jnp.float32)

    r1 = lambda a: a.reshape(1, H)
    cnt2d = lambda cnt: cnt[:N].reshape(N, 1)
    h_u, h_i = _proj(x_user, Wp_user, r1(bp_user), x_item, Wp_item,
                     r1(bp_item))

    agg_i, cnt_i, agg_u, cnt_u = _sc_agg_counts(h_u, h_i, s_ui, d_ui,
                                                s_iu, d_iu, z128, z1, ones1)
    ci, cu = cnt2d(cnt_i), cnt2d(cnt_u)
    h_i, h_u = _layer(agg_i, ci, h_i, Wl0_ui, r1(bl0_ui), Wr0_ui,
                      r1(gamma0_item), r1(beta0_item),
                      agg_u, cu, h_u, Wl0_iu, r1(bl0_iu), Wr0_iu,
                      r1(gamma0_user), r1(beta0_user))

    agg_i, agg_u = _sc_agg_plain(h_u, h_i, s_ui, d_ui, s_iu, d_iu, z128)
    h_i, h_u = _layer(agg_i, ci, h_i, Wl1_ui, r1(bl1_ui), Wr1_ui,
                      r1(gamma1_item), r1(beta1_item),
                      agg_u, cu, h_u, Wl1_iu, r1(bl1_iu), Wr1_iu,
                      r1(gamma1_user), r1(beta1_user))
    return h_u, h_i
